# baseline (speedup 1.0000x reference)
import functools

import jax
import jax.numpy as jnp
from jax import lax
from jax.experimental import pallas as pl
from jax.experimental.pallas import tpu as pltpu
from jax.experimental.pallas import tpu_sc as plsc

FEATURE_DIM = 128
HIDDEN_DIM = 256
EMBED_DIM = 256
N_NODES = 10000
N_NBR = 160000

_SC_ROWS = 64000
_TC_ROWS = N_NBR - _SC_ROWS

_NW = 32
_RPW = _SC_ROWS // _NW
_CHUNK = 40
_NCH = _RPW // _CHUNK

_R_CHUNK = 4000
_N_RSTEPS = _TC_ROWS // _R_CHUNK
_ROW_TILE = 1000
_N_DSTEPS = N_NODES // _ROW_TILE


def _sc_accumulate(buf, rows, accs):
    accs = list(accs)
    ngrp = len(accs)
    for r in range(rows):
        for j in range(ngrp):
            accs[j] = accs[j] + buf[r, pl.ds(16 * j, 16)]
    return tuple(accs)


def _sc_reduce_body(l1_hbm, l2_hbm, out1_hbm, out2_hbm,
                    b1a, b1b, b2a, b2b, acc1_v, acc2_v, sema, semb):
    wid = lax.axis_index("s") * 2 + lax.axis_index("c")
    start = _TC_ROWS + wid * _RPW
    half = _NCH // 2

    def run_array(hbm, bufa, bufb, ngrp):
        accs = tuple(jnp.zeros((16,), jnp.float32) for _ in range(ngrp))
        last = _NCH - 1

        def src(g):
            return hbm.at[pl.ds(start + jnp.minimum(g, last) * _CHUNK, _CHUNK)]

        pltpu.async_copy(src(0), bufa, sema)
        pltpu.async_copy(src(1), bufb, semb)

        def pair_body(h, accs):
            g = 2 * h
            pltpu.make_async_copy(src(g), bufa, sema).wait()
            accs = _sc_accumulate(bufa, _CHUNK, accs)
            pltpu.async_copy(src(g + 2), bufa, sema)
            pltpu.make_async_copy(src(g + 1), bufb, semb).wait()
            accs = _sc_accumulate(bufb, _CHUNK, accs)
            pltpu.async_copy(src(g + 3), bufb, semb)
            return accs

        accs = lax.fori_loop(0, half, pair_body, accs)
        pltpu.make_async_copy(src(last), bufa, sema).wait()
        pltpu.make_async_copy(src(last), bufb, semb).wait()
        return accs

    accs1 = run_array(l1_hbm, b1a, b1b, FEATURE_DIM // 16)
    for j in range(FEATURE_DIM // 16):
        acc1_v[pl.ds(16 * j, 16)] = accs1[j]
    pltpu.sync_copy(acc1_v, out1_hbm.at[pl.ds(wid * FEATURE_DIM, FEATURE_DIM)])

    accs2 = run_array(l2_hbm, b2a, b2b, HIDDEN_DIM // 16)
    for j in range(HIDDEN_DIM // 16):
        acc2_v[pl.ds(16 * j, 16)] = accs2[j]
    pltpu.sync_copy(acc2_v, out2_hbm.at[pl.ds(wid * HIDDEN_DIM, HIDDEN_DIM)])


_sc_reduce = functools.partial(
    pl.kernel,
    mesh=plsc.VectorSubcoreMesh(core_axis_name="c", subcore_axis_name="s"),
    out_type=[
        jax.ShapeDtypeStruct((_NW * FEATURE_DIM,), jnp.float32),
        jax.ShapeDtypeStruct((_NW * HIDDEN_DIM,), jnp.float32),
    ],
    scratch_types=[
        pltpu.VMEM((_CHUNK, FEATURE_DIM), jnp.float32),
        pltpu.VMEM((_CHUNK, FEATURE_DIM), jnp.float32),
        pltpu.VMEM((_CHUNK, HIDDEN_DIM), jnp.float32),
        pltpu.VMEM((_CHUNK, HIDDEN_DIM), jnp.float32),
        pltpu.VMEM((FEATURE_DIM,), jnp.float32),
        pltpu.VMEM((HIDDEN_DIM,), jnp.float32),
        pltpu.SemaphoreType.DMA,
        pltpu.SemaphoreType.DMA,
    ],
)(_sc_reduce_body)


def _tc_reduce_body(l1_ref, l2_ref, p1_ref, p2_ref, acc1_ref, acc2_ref):
    i = pl.program_id(0)

    @pl.when(i == 0)
    def _init():
        acc1_ref[...] = jnp.zeros_like(acc1_ref)
        acc2_ref[...] = jnp.zeros_like(acc2_ref)

    acc1_ref[...] += l1_ref[...].reshape(_R_CHUNK // 8, 8, FEATURE_DIM).sum(axis=0)
    acc2_ref[...] += l2_ref[...].reshape(_R_CHUNK // 8, 8, HIDDEN_DIM).sum(axis=0)

    @pl.when(i == _N_RSTEPS - 1)
    def _finalize():
        p1_ref[...] = acc1_ref[...]
        p2_ref[...] = acc2_ref[...]


def _dense_body(nf_ref, tcp1_ref, tcp2_ref, scp1_ref, scp2_ref,
                wn1_ref, bn1_ref, wn2_ref, bn2_ref,
                ws1_ref, bs1_ref, g1_ref, be1_ref,
                ws2_ref, bs2_ref, g2_ref, be2_ref,
                out_ref, ssq_ref, nbr1_ref, nbr2_ref):
    i = pl.program_id(0)

    @pl.when(i == 0)
    def _init():
        ssq_ref[...] = jnp.zeros_like(ssq_ref)
        agg1 = (tcp1_ref[...].sum(axis=0, keepdims=True)
                + scp1_ref[...].reshape(4, 8, FEATURE_DIM).sum(axis=0)
                .sum(axis=0, keepdims=True)) * (1.0 / N_NBR)
        agg2 = (tcp2_ref[...].sum(axis=0, keepdims=True)
                + scp2_ref[...].reshape(4, 8, HIDDEN_DIM).sum(axis=0)
                .sum(axis=0, keepdims=True)) * (1.0 / N_NBR)
        nbr1_ref[...] = jnp.dot(agg1, wn1_ref[...],
                                preferred_element_type=jnp.float32) + bn1_ref[...]
        nbr2_ref[...] = jnp.dot(agg2, wn2_ref[...],
                                preferred_element_type=jnp.float32) + bn2_ref[...]

    @pl.when(i < _N_DSTEPS)
    def _compute():
        x = nf_ref[...]
        out1 = (jnp.dot(x, ws1_ref[...], preferred_element_type=jnp.float32)
                + bs1_ref[...] + nbr1_ref[...])
        mu1 = jnp.mean(out1, axis=-1, keepdims=True)
        d1 = out1 - mu1
        var1 = jnp.mean(d1 * d1, axis=-1, keepdims=True)
        h1 = jnp.maximum(
            d1 * lax.rsqrt(var1 + 1e-5) * g1_ref[...] + be1_ref[...], 0.0)
        out2 = (jnp.dot(h1, ws2_ref[...], preferred_element_type=jnp.float32)
                + bs2_ref[...] + nbr2_ref[...])
        mu2 = jnp.mean(out2, axis=-1, keepdims=True)
        d2 = out2 - mu2
        var2 = jnp.mean(d2 * d2, axis=-1, keepdims=True)
        h2 = jnp.maximum(
            d2 * lax.rsqrt(var2 + 1e-5) * g2_ref[...] + be2_ref[...], 0.0)
        out_ref[pl.ds(i * _ROW_TILE, _ROW_TILE), :] = h2
        ssq_ref[...] += jnp.sum(h2 * h2, axis=0, keepdims=True)

    @pl.when(i == _N_DSTEPS)
    def _normalize():
        scale = 1.0 / jnp.maximum(jnp.sqrt(ssq_ref[...]), 1e-12)
        out_ref[...] = out_ref[...] * scale


def kernel(node_feat, neighbor_feats_l1, neighbor_feats_l2, W_self1, b_self1,
           W_nbr1, b_nbr1, g1, be1, W_self2, b_self2, W_nbr2, b_nbr2, g2, be2):
    f32 = jnp.float32
    row = lambda v: v.reshape(1, -1)

    sc_p1, sc_p2 = _sc_reduce(neighbor_feats_l1, neighbor_feats_l2)
    sc_p1 = sc_p1.reshape(_NW, FEATURE_DIM)
    sc_p2 = sc_p2.reshape(_NW, HIDDEN_DIM)

    tc_p1, tc_p2 = pl.pallas_call(
        _tc_reduce_body,
        grid=(_N_RSTEPS,),
        in_specs=[
            pl.BlockSpec((_R_CHUNK, FEATURE_DIM), lambda i: (i, 0)),
            pl.BlockSpec((_R_CHUNK, HIDDEN_DIM), lambda i: (i, 0)),
        ],
        out_specs=[
            pl.BlockSpec((8, FEATURE_DIM), lambda i: (0, 0)),
            pl.BlockSpec((8, HIDDEN_DIM), lambda i: (0, 0)),
        ],
        out_shape=[
            jax.ShapeDtypeStruct((8, FEATURE_DIM), f32),
            jax.ShapeDtypeStruct((8, HIDDEN_DIM), f32),
        ],
        scratch_shapes=[
            pltpu.VMEM((8, FEATURE_DIM), f32),
            pltpu.VMEM((8, HIDDEN_DIM), f32),
        ],
    )(neighbor_feats_l1, neighbor_feats_l2)

    h2 = pl.pallas_call(
        _dense_body,
        grid=(_N_DSTEPS + 1,),
        in_specs=[
            pl.BlockSpec((_ROW_TILE, FEATURE_DIM),
                         lambda i: (jnp.minimum(i, _N_DSTEPS - 1), 0)),
            pl.BlockSpec((8, FEATURE_DIM), lambda i: (0, 0)),
            pl.BlockSpec((8, HIDDEN_DIM), lambda i: (0, 0)),
            pl.BlockSpec((_NW, FEATURE_DIM), lambda i: (0, 0)),
            pl.BlockSpec((_NW, HIDDEN_DIM), lambda i: (0, 0)),
            pl.BlockSpec((FEATURE_DIM, HIDDEN_DIM), lambda i: (0, 0)),
            pl.BlockSpec((1, HIDDEN_DIM), lambda i: (0, 0)),
            pl.BlockSpec((HIDDEN_DIM, EMBED_DIM), lambda i: (0, 0)),
            pl.BlockSpec((1, EMBED_DIM), lambda i: (0, 0)),
            pl.BlockSpec((FEATURE_DIM, HIDDEN_DIM), lambda i: (0, 0)),
            pl.BlockSpec((1, HIDDEN_DIM), lambda i: (0, 0)),
            pl.BlockSpec((1, HIDDEN_DIM), lambda i: (0, 0)),
            pl.BlockSpec((1, HIDDEN_DIM), lambda i: (0, 0)),
            pl.BlockSpec((HIDDEN_DIM, EMBED_DIM), lambda i: (0, 0)),
            pl.BlockSpec((1, EMBED_DIM), lambda i: (0, 0)),
            pl.BlockSpec((1, EMBED_DIM), lambda i: (0, 0)),
            pl.BlockSpec((1, EMBED_DIM), lambda i: (0, 0)),
        ],
        out_specs=pl.BlockSpec((N_NODES, EMBED_DIM), lambda i: (0, 0)),
        out_shape=jax.ShapeDtypeStruct((N_NODES, EMBED_DIM), f32),
        scratch_shapes=[
            pltpu.VMEM((1, EMBED_DIM), f32),
            pltpu.VMEM((1, HIDDEN_DIM), f32),
            pltpu.VMEM((1, EMBED_DIM), f32),
        ],
    )(node_feat, tc_p1, tc_p2, sc_p1, sc_p2,
      W_nbr1, row(b_nbr1), W_nbr2, row(b_nbr2),
      W_self1, row(b_self1), row(g1), row(be1),
      W_self2, row(b_self2), row(g2), row(be2))

    return h2


# speedup vs baseline: 2.3399x; 2.3399x over previous
import jax
import jax.numpy as jnp
from jax import lax
from jax.experimental import pallas as pl
from jax.experimental.pallas import tpu as pltpu

FEATURE_DIM = 128
HIDDEN_DIM = 256
EMBED_DIM = 256
N_NODES = 10000
N_NBR = 160000

_R_CHUNK = 4000
_N_RSTEPS = N_NBR // _R_CHUNK
_ROW_TILE = 1000
_N_DSTEPS = N_NODES // _ROW_TILE


def _body(l1_ref, l2_ref, nf_ref,
          wn1_ref, bn1_ref, wn2_ref, bn2_ref,
          ws1_ref, bs1_ref, g1_ref, be1_ref,
          ws2_ref, bs2_ref, g2_ref, be2_ref,
          out_ref, acc1_ref, acc2_ref, a_ref, ssq_ref, nbr1_ref, nbr2_ref):
    i = pl.program_id(0)

    @pl.when(i == 0)
    def _init():
        acc1_ref[...] = jnp.zeros_like(acc1_ref)
        acc2_ref[...] = jnp.zeros_like(acc2_ref)
        ssq_ref[...] = jnp.zeros_like(ssq_ref)

    @pl.when(i < _N_RSTEPS)
    def _reduce():
        acc1_ref[...] += l1_ref[...].reshape(_R_CHUNK // 8, 8, FEATURE_DIM).sum(axis=0)
        acc2_ref[...] += l2_ref[...].reshape(_R_CHUNK // 8, 8, HIDDEN_DIM).sum(axis=0)

    @pl.when(i < _N_DSTEPS)
    def _precompute_a():
        a_ref[pl.ds(i * _ROW_TILE, _ROW_TILE), :] = (
            jnp.dot(nf_ref[...], ws1_ref[...],
                    preferred_element_type=jnp.float32) + bs1_ref[...])

    @pl.when(i == _N_RSTEPS - 1)
    def _finalize_aggs():
        agg1 = acc1_ref[...].sum(axis=0, keepdims=True) * (1.0 / N_NBR)
        agg2 = acc2_ref[...].sum(axis=0, keepdims=True) * (1.0 / N_NBR)
        nbr1_ref[...] = jnp.dot(agg1, wn1_ref[...],
                                preferred_element_type=jnp.float32) + bn1_ref[...]
        nbr2_ref[...] = jnp.dot(agg2, wn2_ref[...],
                                preferred_element_type=jnp.float32) + bn2_ref[...]

    @pl.when(jnp.logical_and(i >= _N_RSTEPS, i < _N_RSTEPS + _N_DSTEPS))
    def _dense():
        j = i - _N_RSTEPS
        out1 = a_ref[pl.ds(j * _ROW_TILE, _ROW_TILE), :] + nbr1_ref[...]
        mu1 = jnp.mean(out1, axis=-1, keepdims=True)
        d1 = out1 - mu1
        var1 = jnp.mean(d1 * d1, axis=-1, keepdims=True)
        h1 = jnp.maximum(
            d1 * lax.rsqrt(var1 + 1e-5) * g1_ref[...] + be1_ref[...], 0.0)
        out2 = (jnp.dot(h1, ws2_ref[...], preferred_element_type=jnp.float32)
                + bs2_ref[...] + nbr2_ref[...])
        mu2 = jnp.mean(out2, axis=-1, keepdims=True)
        d2 = out2 - mu2
        var2 = jnp.mean(d2 * d2, axis=-1, keepdims=True)
        h2 = jnp.maximum(
            d2 * lax.rsqrt(var2 + 1e-5) * g2_ref[...] + be2_ref[...], 0.0)
        out_ref[pl.ds(j * _ROW_TILE, _ROW_TILE), :] = h2
        ssq_ref[...] += jnp.sum(h2 * h2, axis=0, keepdims=True)

    @pl.when(i == _N_RSTEPS + _N_DSTEPS)
    def _normalize():
        scale = 1.0 / jnp.maximum(jnp.sqrt(ssq_ref[...]), 1e-12)
        out_ref[...] = out_ref[...] * scale


def kernel(node_feat, neighbor_feats_l1, neighbor_feats_l2, W_self1, b_self1,
           W_nbr1, b_nbr1, g1, be1, W_self2, b_self2, W_nbr2, b_nbr2, g2, be2):
    f32 = jnp.float32
    row = lambda v: v.reshape(1, -1)
    n_steps = _N_RSTEPS + _N_DSTEPS + 1
    last_r = _N_RSTEPS - 1
    last_d = _N_DSTEPS - 1

    h2 = pl.pallas_call(
        _body,
        grid=(n_steps,),
        in_specs=[
            pl.BlockSpec((_R_CHUNK, FEATURE_DIM),
                         lambda i: (jnp.minimum(i, last_r), 0)),
            pl.BlockSpec((_R_CHUNK, HIDDEN_DIM),
                         lambda i: (jnp.minimum(i, last_r), 0)),
            pl.BlockSpec((_ROW_TILE, FEATURE_DIM),
                         lambda i: (jnp.minimum(i, last_d), 0)),
            pl.BlockSpec((FEATURE_DIM, HIDDEN_DIM), lambda i: (0, 0)),
            pl.BlockSpec((1, HIDDEN_DIM), lambda i: (0, 0)),
            pl.BlockSpec((HIDDEN_DIM, EMBED_DIM), lambda i: (0, 0)),
            pl.BlockSpec((1, EMBED_DIM), lambda i: (0, 0)),
            pl.BlockSpec((FEATURE_DIM, HIDDEN_DIM), lambda i: (0, 0)),
            pl.BlockSpec((1, HIDDEN_DIM), lambda i: (0, 0)),
            pl.BlockSpec((1, HIDDEN_DIM), lambda i: (0, 0)),
            pl.BlockSpec((1, HIDDEN_DIM), lambda i: (0, 0)),
            pl.BlockSpec((HIDDEN_DIM, EMBED_DIM), lambda i: (0, 0)),
            pl.BlockSpec((1, EMBED_DIM), lambda i: (0, 0)),
            pl.BlockSpec((1, EMBED_DIM), lambda i: (0, 0)),
            pl.BlockSpec((1, EMBED_DIM), lambda i: (0, 0)),
        ],
        out_specs=pl.BlockSpec((N_NODES, EMBED_DIM), lambda i: (0, 0)),
        out_shape=jax.ShapeDtypeStruct((N_NODES, EMBED_DIM), f32),
        scratch_shapes=[
            pltpu.VMEM((8, FEATURE_DIM), f32),
            pltpu.VMEM((8, HIDDEN_DIM), f32),
            pltpu.VMEM((N_NODES, HIDDEN_DIM), f32),
            pltpu.VMEM((1, EMBED_DIM), f32),
            pltpu.VMEM((1, HIDDEN_DIM), f32),
            pltpu.VMEM((1, EMBED_DIM), f32),
        ],
    )(neighbor_feats_l1, neighbor_feats_l2, node_feat,
      W_nbr1, row(b_nbr1), W_nbr2, row(b_nbr2),
      W_self1, row(b_self1), row(g1), row(be1),
      W_self2, row(b_self2), row(g2), row(be2))

    return h2


# speedup vs baseline: 2.3769x; 1.0158x over previous
import jax
import jax.numpy as jnp
from jax import lax
from jax.experimental import pallas as pl
from jax.experimental.pallas import tpu as pltpu

FEATURE_DIM = 128
HIDDEN_DIM = 256
EMBED_DIM = 256
N_NODES = 10000
N_NBR = 160000

_R_CHUNK = 8000
_N_RSTEPS = N_NBR // _R_CHUNK
_ROW_TILE = 1000
_N_DSTEPS = N_NODES // _ROW_TILE


def _body(l1_ref, l2_ref, nf_ref,
          wn1_ref, bn1_ref, wn2_ref, bn2_ref,
          ws1_ref, bs1_ref, g1_ref, be1_ref,
          ws2_ref, bs2_ref, g2_ref, be2_ref,
          out_ref, acc1_ref, acc2_ref, a_ref, ssq_ref, nbr1_ref, nbr2_ref):
    i = pl.program_id(0)

    @pl.when(i == 0)
    def _init():
        acc1_ref[...] = jnp.zeros_like(acc1_ref)
        acc2_ref[...] = jnp.zeros_like(acc2_ref)
        ssq_ref[...] = jnp.zeros_like(ssq_ref)

    @pl.when(i < _N_RSTEPS)
    def _reduce():
        acc1_ref[...] += l1_ref[...].reshape(_R_CHUNK // 8, 8, FEATURE_DIM).sum(axis=0)
        acc2_ref[...] += l2_ref[...].reshape(_R_CHUNK // 8, 8, HIDDEN_DIM).sum(axis=0)

    @pl.when(i < _N_DSTEPS)
    def _precompute_a():
        a_ref[pl.ds(i * _ROW_TILE, _ROW_TILE), :] = (
            jnp.dot(nf_ref[...], ws1_ref[...],
                    preferred_element_type=jnp.float32) + bs1_ref[...])

    @pl.when(i == _N_RSTEPS - 1)
    def _finalize_aggs():
        agg1 = acc1_ref[...].sum(axis=0, keepdims=True) * (1.0 / N_NBR)
        agg2 = acc2_ref[...].sum(axis=0, keepdims=True) * (1.0 / N_NBR)
        nbr1_ref[...] = jnp.dot(agg1, wn1_ref[...],
                                preferred_element_type=jnp.float32) + bn1_ref[...]
        nbr2_ref[...] = jnp.dot(agg2, wn2_ref[...],
                                preferred_element_type=jnp.float32) + bn2_ref[...]

    @pl.when(jnp.logical_and(i >= _N_RSTEPS, i < _N_RSTEPS + _N_DSTEPS))
    def _dense():
        j = i - _N_RSTEPS
        out1 = a_ref[pl.ds(j * _ROW_TILE, _ROW_TILE), :] + nbr1_ref[...]
        mu1 = jnp.mean(out1, axis=-1, keepdims=True)
        d1 = out1 - mu1
        var1 = jnp.mean(d1 * d1, axis=-1, keepdims=True)
        h1 = jnp.maximum(
            d1 * lax.rsqrt(var1 + 1e-5) * g1_ref[...] + be1_ref[...], 0.0)
        out2 = (jnp.dot(h1, ws2_ref[...], preferred_element_type=jnp.float32)
                + bs2_ref[...] + nbr2_ref[...])
        mu2 = jnp.mean(out2, axis=-1, keepdims=True)
        d2 = out2 - mu2
        var2 = jnp.mean(d2 * d2, axis=-1, keepdims=True)
        h2 = jnp.maximum(
            d2 * lax.rsqrt(var2 + 1e-5) * g2_ref[...] + be2_ref[...], 0.0)
        out_ref[pl.ds(j * _ROW_TILE, _ROW_TILE), :] = h2
        ssq_ref[...] += jnp.sum(h2 * h2, axis=0, keepdims=True)

    @pl.when(i == _N_RSTEPS + _N_DSTEPS)
    def _normalize():
        scale = 1.0 / jnp.maximum(jnp.sqrt(ssq_ref[...]), 1e-12)
        out_ref[...] = out_ref[...] * scale


def kernel(node_feat, neighbor_feats_l1, neighbor_feats_l2, W_self1, b_self1,
           W_nbr1, b_nbr1, g1, be1, W_self2, b_self2, W_nbr2, b_nbr2, g2, be2):
    f32 = jnp.float32
    row = lambda v: v.reshape(1, -1)
    n_steps = _N_RSTEPS + _N_DSTEPS + 1
    last_r = _N_RSTEPS - 1
    last_d = _N_DSTEPS - 1

    h2 = pl.pallas_call(
        _body,
        grid=(n_steps,),
        in_specs=[
            pl.BlockSpec((_R_CHUNK, FEATURE_DIM),
                         lambda i: (jnp.minimum(i, last_r), 0)),
            pl.BlockSpec((_R_CHUNK, HIDDEN_DIM),
                         lambda i: (jnp.minimum(i, last_r), 0)),
            pl.BlockSpec((_ROW_TILE, FEATURE_DIM),
                         lambda i: (jnp.minimum(i, last_d), 0)),
            pl.BlockSpec((FEATURE_DIM, HIDDEN_DIM), lambda i: (0, 0)),
            pl.BlockSpec((1, HIDDEN_DIM), lambda i: (0, 0)),
            pl.BlockSpec((HIDDEN_DIM, EMBED_DIM), lambda i: (0, 0)),
            pl.BlockSpec((1, EMBED_DIM), lambda i: (0, 0)),
            pl.BlockSpec((FEATURE_DIM, HIDDEN_DIM), lambda i: (0, 0)),
            pl.BlockSpec((1, HIDDEN_DIM), lambda i: (0, 0)),
            pl.BlockSpec((1, HIDDEN_DIM), lambda i: (0, 0)),
            pl.BlockSpec((1, HIDDEN_DIM), lambda i: (0, 0)),
            pl.BlockSpec((HIDDEN_DIM, EMBED_DIM), lambda i: (0, 0)),
            pl.BlockSpec((1, EMBED_DIM), lambda i: (0, 0)),
            pl.BlockSpec((1, EMBED_DIM), lambda i: (0, 0)),
            pl.BlockSpec((1, EMBED_DIM), lambda i: (0, 0)),
        ],
        out_specs=pl.BlockSpec((N_NODES, EMBED_DIM), lambda i: (0, 0)),
        out_shape=jax.ShapeDtypeStruct((N_NODES, EMBED_DIM), f32),
        scratch_shapes=[
            pltpu.VMEM((8, FEATURE_DIM), f32),
            pltpu.VMEM((8, HIDDEN_DIM), f32),
            pltpu.VMEM((N_NODES, HIDDEN_DIM), f32),
            pltpu.VMEM((1, EMBED_DIM), f32),
            pltpu.VMEM((1, HIDDEN_DIM), f32),
            pltpu.VMEM((1, EMBED_DIM), f32),
        ],
    )(neighbor_feats_l1, neighbor_feats_l2, node_feat,
      W_nbr1, row(b_nbr1), W_nbr2, row(b_nbr2),
      W_self1, row(b_self1), row(g1), row(be1),
      W_self2, row(b_self2), row(g2), row(be2))

    return h2
